# Initial kernel scaffold; baseline (speedup 1.0000x reference)
import jax
import jax.numpy as jnp
from jax.experimental import pallas as pl


def kernel(x, pos, edge_index, W0, b0, p1, W1, b1, p2, W2, b2, Wu0, bu0, Wu1, bu1):
    raise NotImplementedError("write your pallas kernel here")


# speedup vs baseline: 11.6495x; 11.6495x over previous
import functools
import math

import jax
import jax.numpy as jnp
from jax import lax
from jax.experimental import pallas as pl
from jax.experimental.pallas import tpu as pltpu
from jax.experimental.pallas import tpu_sc as plsc

N = 10000
E = 160000
NUM_CLASSES = 50

N_PAD = 10240
ZROW = N
E_PAD = 163840
EDGE_ROWS = E_PAD // 128
ROWS_PER_WORKER = EDGE_ROWS // 32
STRIPE = N_PAD // 16


@functools.lru_cache(maxsize=None)
def _sc_agg(F):
    mesh = plsc.VectorSubcoreMesh(core_axis_name="c", subcore_axis_name="s")

    @functools.partial(
        pl.kernel,
        mesh=mesh,
        compiler_params=pltpu.CompilerParams(use_tc_tiling_on_sc=False),
        out_type=jax.ShapeDtypeStruct((2, N_PAD, F), jnp.float32),
        scratch_types=[
            pltpu.VMEM((8, 128), jnp.int32),
            pltpu.VMEM((8, 128), jnp.int32),
            pltpu.VMEM((128, F), jnp.float32),
            pltpu.VMEM_SHARED((N_PAD, F), jnp.float32),
            pltpu.SemaphoreType.DMA,
        ],
    )
    def agg(table_h, src_h, dst_h, zeros_h, out_h,
            srcb, dstb, rows, acc, sem):
        c = lax.axis_index("c")
        s = lax.axis_index("s")
        w = c * 16 + s
        pltpu.sync_copy(zeros_h.at[pl.ds(s * STRIPE, STRIPE)],
                        acc.at[pl.ds(s * STRIPE, STRIPE)])
        plsc.subcore_barrier()

        row0 = w * ROWS_PER_WORKER

        def outer(i, carry):
            base = row0 + i * 8
            pltpu.sync_copy(src_h.at[pl.ds(base, 8)], srcb)
            pltpu.sync_copy(dst_h.at[pl.ds(base, 8)], dstb)
            for r in range(8):
                pltpu.async_copy(table_h.at[srcb.at[r]], rows, sem).wait()
                pltpu.sync_copy(rows, acc.at[dstb.at[r]], add=True)
            return carry

        lax.fori_loop(0, ROWS_PER_WORKER // 8, outer, 0)
        plsc.subcore_barrier()
        pltpu.sync_copy(acc.at[pl.ds(s * STRIPE, STRIPE)],
                        out_h.at[c].at[pl.ds(s * STRIPE, STRIPE)])

    return agg


def _prep_body(x_ref, w_ref, degp_ref, scale_ref, m_ref, hp_ref, dinv_ref):
    deg = degp_ref[0, :, 0:1] + degp_ref[1, :, 0:1] + 2.0
    dinv = lax.rsqrt(deg)
    xe = x_ref[...] * scale_ref[...]
    hp = jnp.dot(xe, w_ref[...], preferred_element_type=jnp.float32) * dinv
    hp_ref[...] = hp * m_ref[...]
    dinv_ref[...] = dinv


_ROWB = 2560


def _tc_prep(X, W, degp, scale, mask):
    n, fin = X.shape
    fout = W.shape[1]
    g = n // _ROWB
    return pl.pallas_call(
        _prep_body,
        grid=(g,),
        in_specs=[
            pl.BlockSpec((_ROWB, fin), lambda i: (i, 0)),
            pl.BlockSpec((fin, fout), lambda i: (0, 0)),
            pl.BlockSpec((2, _ROWB, 16), lambda i: (0, i, 0)),
            pl.BlockSpec((_ROWB, 1), lambda i: (i, 0)),
            pl.BlockSpec((_ROWB, 1), lambda i: (i, 0)),
        ],
        out_specs=(pl.BlockSpec((_ROWB, fout), lambda i: (i, 0)),
                   pl.BlockSpec((_ROWB, 1), lambda i: (i, 0))),
        out_shape=(jax.ShapeDtypeStruct((n, fout), jnp.float32),
                   jax.ShapeDtypeStruct((n, 1), jnp.float32)),
    )(X, W, degp, scale, mask)


def _finish_body(relu, p_ref, hp_ref, dinv_ref, b_ref, rm_ref, y_ref):
    psum = p_ref[0, :, :] + p_ref[1, :, :]
    dinv = dinv_ref[...]
    y = dinv * psum + 2.0 * dinv * hp_ref[...] + b_ref[...]
    if relu:
        y = jnp.maximum(y, 0.0)
    y_ref[...] = y * rm_ref[...]


def _rowwise_specs(f):
    return [
        pl.BlockSpec((2, _ROWB, f), lambda i: (0, i, 0)),
        pl.BlockSpec((_ROWB, f), lambda i: (i, 0)),
        pl.BlockSpec((_ROWB, 1), lambda i: (i, 0)),
        pl.BlockSpec((1, f), lambda i: (0, 0)),
        pl.BlockSpec((_ROWB, 1), lambda i: (i, 0)),
    ]


def _tc_finish(P, hp, dinv, b, rowmask, relu):
    n, f = hp.shape
    return pl.pallas_call(
        functools.partial(_finish_body, relu),
        grid=(n // _ROWB,),
        in_specs=_rowwise_specs(f),
        out_specs=pl.BlockSpec((_ROWB, f), lambda i: (i, 0)),
        out_shape=jax.ShapeDtypeStruct((n, f), jnp.float32),
    )(P, hp, dinv, b.reshape(1, f), rowmask)


def _logsoftmax_body(p_ref, hp_ref, dinv_ref, b_ref, cm_ref, y_ref):
    psum = p_ref[0, :, :] + p_ref[1, :, :]
    dinv = dinv_ref[...]
    y = dinv * psum + 2.0 * dinv * hp_ref[...] + b_ref[...]
    cm = cm_ref[...]
    yl = jnp.where(cm > 0, y, -1e30)
    mx = jnp.max(yl, axis=1, keepdims=True)
    ex = jnp.where(cm > 0, jnp.exp(yl - mx), 0.0)
    denom = jnp.sum(ex, axis=1, keepdims=True)
    y_ref[...] = (yl - mx) - jnp.log(denom)


def _tc_logsoftmax(P, hp, dinv, b, colmask):
    n, f = hp.shape
    specs = _rowwise_specs(f)
    specs[-1] = pl.BlockSpec((1, f), lambda i: (0, 0))
    return pl.pallas_call(
        _logsoftmax_body,
        grid=(n // _ROWB,),
        in_specs=specs,
        out_specs=pl.BlockSpec((_ROWB, f), lambda i: (i, 0)),
        out_shape=jax.ShapeDtypeStruct((n, f), jnp.float32),
    )(P, hp, dinv, b.reshape(1, f), colmask)


def _score_body(k, y_ref, p_ref, valid_ref, m_ref, s_ref):
    pv = p_ref[...]
    pn = jnp.sqrt(jnp.sum(pv * pv))
    st = jnp.tanh(jnp.dot(y_ref[...], pv,
                          preferred_element_type=jnp.float32) / pn)
    valid = valid_ref[...]
    sm = jnp.where(valid > 0, st, -3.0)
    bi = lax.bitcast_convert_type(sm, jnp.int32)
    mag = jnp.bitwise_and(bi, jnp.int32(0x7FFFFFFF))
    key = jnp.where(bi >= 0, bi, -mag)

    kf = jnp.float32(k)

    def cnt_ge(t):
        return jnp.sum(jnp.where(key >= t, 1.0, 0.0))

    def step(_, lh):
        lo, hi = lh
        mid = lo + (hi - lo + 1) // 2
        ok = cnt_ge(mid) >= kf
        return (jnp.where(ok, mid, lo), jnp.where(ok, hi, mid - 1))

    lo0 = jnp.int32(-0x40400001)
    hi0 = jnp.int32(0x3F800000)
    t, _ = lax.fori_loop(0, 32, step, (lo0, hi0))

    gt = key > t
    eq = key == t
    needed = kf - jnp.sum(jnp.where(gt, 1.0, 0.0))
    idx = lax.broadcasted_iota(jnp.int32, key.shape, 0)

    def cnt_eq_lt(T):
        return jnp.sum(jnp.where(eq & (idx < T), 1.0, 0.0))

    def step2(_, lh):
        lo, hi = lh
        mid = lo + (hi - lo + 1) // 2
        ok = cnt_eq_lt(mid) <= needed
        return (jnp.where(ok, mid, lo), jnp.where(ok, hi, mid - 1))

    T, _ = lax.fori_loop(0, 16, step2, (jnp.int32(0), jnp.int32(N_PAD)))

    sel = gt | (eq & (idx < T))
    m_ref[...] = jnp.where(sel, 1.0, 0.0)
    s_ref[...] = st


def _tc_score(Y, p, valid, k):
    n, f = Y.shape
    return pl.pallas_call(
        functools.partial(_score_body, k),
        out_shape=(jax.ShapeDtypeStruct((n, 1), jnp.float32),
                   jax.ShapeDtypeStruct((n, 1), jnp.float32)),
    )(Y, p.reshape(f, 1), valid)


def _pad_rows(a, rows):
    return jnp.pad(a, ((0, rows - a.shape[0]),) + ((0, 0),) * (a.ndim - 1))


def kernel(x, pos, edge_index, W0, b0, p1, W1, b1, p2, W2, b2,
           Wu0, bu0, Wu1, bu1):
    f32 = jnp.float32

    x0 = _pad_rows(jnp.concatenate([x, pos], axis=-1), N_PAD)
    x0 = jnp.pad(x0, ((0, 0), (0, 16 - 9)))
    W0p = jnp.pad(W0, ((0, 16 - 9), (0, 0)))
    Wu1p = jnp.pad(Wu1, ((0, 0), (0, 64 - NUM_CLASSES)))
    bu1p = jnp.pad(bu1, (0, 64 - NUM_CLASSES))

    src = jnp.full((E_PAD,), ZROW, jnp.int32).at[:E].set(edge_index[0])
    dst = jnp.full((E_PAD,), ZROW, jnp.int32).at[:E].set(edge_index[1])
    src = src.reshape(EDGE_ROWS, 128)
    dst = dst.reshape(EDGE_ROWS, 128)

    rowmask = (jnp.arange(N_PAD) < N).astype(f32).reshape(N_PAD, 1)
    z16 = jnp.zeros((N_PAD, 16), f32)
    z64 = jnp.zeros((N_PAD, 64), f32)
    z128 = jnp.zeros((N_PAD, 128), f32)
    onescale = jnp.ones((N_PAD, 1), f32)
    colmask = (jnp.arange(64) < NUM_CLASSES).astype(f32).reshape(1, 64)

    k1 = int(math.ceil(0.8 * N))
    k2 = int(math.ceil(0.8 * k1))

    agg16 = _sc_agg(16)
    agg64 = _sc_agg(64)
    agg128 = _sc_agg(128)

    def ones_table(m):
        return jnp.broadcast_to(m, (N_PAD, 16))

    deg0 = agg16(ones_table(rowmask), src, dst, z16)
    hp0, dinv0 = _tc_prep(x0, W0p, deg0, onescale, rowmask)
    P0 = agg64(hp0, src, dst, z64)
    h0 = _tc_finish(P0, hp0, dinv0, b0, rowmask, relu=True)

    m1, s1 = _tc_score(h0, p1, rowmask, k1)
    deg1 = agg16(ones_table(m1), src, dst, z16)
    hp1, dinv1 = _tc_prep(h0, W1, deg1, s1, m1)
    P1 = agg128(hp1, src, dst, z128)
    h1 = _tc_finish(P1, hp1, dinv1, b1, rowmask, relu=True)

    m2, s2 = _tc_score(h1, p2, m1, k2)
    deg2 = agg16(ones_table(m2), src, dst, z16)
    hp2, dinv2 = _tc_prep(h1, W2, deg2, s2, m2)
    P2a = agg128(hp2[:, :128], src, dst, z128)
    P2b = agg128(hp2[:, 128:], src, dst, z128)
    P2 = jnp.concatenate([P2a, P2b], axis=2)
    h2 = _tc_finish(P2, hp2, dinv2, b2, m2, relu=True)

    u1in = jnp.concatenate([h1, h2], axis=-1)
    hpu0, _ = _tc_prep(u1in, Wu0, deg1, onescale, m1)
    Pu0 = agg128(hpu0, src, dst, z128)
    u1 = _tc_finish(Pu0, hpu0, dinv1, bu0, m1, relu=True)

    oin = jnp.concatenate([h0, u1], axis=-1)
    hpu1, _ = _tc_prep(oin, Wu1p, deg0, onescale, rowmask)
    Pu1 = agg64(hpu1, src, dst, z64)
    out = _tc_logsoftmax(Pu1, hpu1, dinv0, bu1p, colmask)

    return out[:N, :NUM_CLASSES]


# speedup vs baseline: 13.7306x; 1.1786x over previous
import functools
import math

import jax
import jax.numpy as jnp
from jax import lax
from jax.experimental import pallas as pl
from jax.experimental.pallas import tpu as pltpu
from jax.experimental.pallas import tpu_sc as plsc

N = 10000
E = 160000
NUM_CLASSES = 50

N_PAD = 10240
ZROW = N
E_PAD = 163840
EDGE_ROWS = E_PAD // 128
ROWS_PER_WORKER = EDGE_ROWS // 32
STRIPE = N_PAD // 16


@functools.lru_cache(maxsize=None)
def _sc_agg(F):
    mesh = plsc.VectorSubcoreMesh(core_axis_name="c", subcore_axis_name="s")

    @functools.partial(
        pl.kernel,
        mesh=mesh,
        compiler_params=pltpu.CompilerParams(use_tc_tiling_on_sc=False),
        out_type=jax.ShapeDtypeStruct((2, N_PAD, F), jnp.float32),
        scratch_types=[
            pltpu.VMEM((ROWS_PER_WORKER, 128), jnp.int32),
            pltpu.VMEM((ROWS_PER_WORKER, 128), jnp.int32),
            pltpu.VMEM((128, F), jnp.float32),
            pltpu.VMEM((128, F), jnp.float32),
            pltpu.VMEM_SHARED((N_PAD, F), jnp.float32),
            pltpu.SemaphoreType.DMA,
            pltpu.SemaphoreType.DMA,
        ],
    )
    def agg(table_h, src_h, dst_h, zeros_h, out_h,
            srcb, dstb, rows0, rows1, acc, sem0, sem1):
        c = lax.axis_index("c")
        s = lax.axis_index("s")
        w = c * 16 + s
        pltpu.sync_copy(zeros_h.at[pl.ds(s * STRIPE, STRIPE)],
                        acc.at[pl.ds(s * STRIPE, STRIPE)])
        plsc.subcore_barrier()

        row0 = w * ROWS_PER_WORKER
        pltpu.sync_copy(src_h.at[pl.ds(row0, ROWS_PER_WORKER)], srcb)
        pltpu.sync_copy(dst_h.at[pl.ds(row0, ROWS_PER_WORKER)], dstb)

        pltpu.async_copy(table_h.at[srcb.at[0]], rows0, sem0)
        pltpu.async_copy(table_h.at[srcb.at[1]], rows1, sem1)

        def body(t, carry):
            j0 = 2 * t
            pltpu.make_async_copy(table_h.at[srcb.at[0]], rows0, sem0).wait()
            pltpu.sync_copy(rows0, acc.at[dstb.at[j0]], add=True)

            @pl.when(j0 + 2 < ROWS_PER_WORKER)
            def _():
                pltpu.async_copy(table_h.at[srcb.at[j0 + 2]], rows0, sem0)

            pltpu.make_async_copy(table_h.at[srcb.at[1]], rows1, sem1).wait()
            pltpu.sync_copy(rows1, acc.at[dstb.at[j0 + 1]], add=True)

            @pl.when(j0 + 3 < ROWS_PER_WORKER)
            def _():
                pltpu.async_copy(table_h.at[srcb.at[j0 + 3]], rows1, sem1)

            return carry

        lax.fori_loop(0, ROWS_PER_WORKER // 2, body, 0)
        plsc.subcore_barrier()
        pltpu.sync_copy(acc.at[pl.ds(s * STRIPE, STRIPE)],
                        out_h.at[c].at[pl.ds(s * STRIPE, STRIPE)])

    return agg


def _prep_body(x_ref, w_ref, degp_ref, scale_ref, m_ref, hp_ref, dinv_ref):
    deg = degp_ref[0, :, 0:1] + degp_ref[1, :, 0:1] + 2.0
    dinv = lax.rsqrt(deg)
    xe = x_ref[...] * scale_ref[...]
    hp = jnp.dot(xe, w_ref[...], preferred_element_type=jnp.float32) * dinv
    hp_ref[...] = hp * m_ref[...]
    dinv_ref[...] = dinv


_ROWB = 2560


def _tc_prep(X, W, degp, scale, mask):
    n, fin = X.shape
    fout = W.shape[1]
    g = n // _ROWB
    return pl.pallas_call(
        _prep_body,
        grid=(g,),
        in_specs=[
            pl.BlockSpec((_ROWB, fin), lambda i: (i, 0)),
            pl.BlockSpec((fin, fout), lambda i: (0, 0)),
            pl.BlockSpec((2, _ROWB, 16), lambda i: (0, i, 0)),
            pl.BlockSpec((_ROWB, 1), lambda i: (i, 0)),
            pl.BlockSpec((_ROWB, 1), lambda i: (i, 0)),
        ],
        out_specs=(pl.BlockSpec((_ROWB, fout), lambda i: (i, 0)),
                   pl.BlockSpec((_ROWB, 1), lambda i: (i, 0))),
        out_shape=(jax.ShapeDtypeStruct((n, fout), jnp.float32),
                   jax.ShapeDtypeStruct((n, 1), jnp.float32)),
    )(X, W, degp, scale, mask)


def _finish_body(relu, p_ref, hp_ref, dinv_ref, b_ref, rm_ref, y_ref):
    psum = p_ref[0, :, :] + p_ref[1, :, :]
    dinv = dinv_ref[...]
    y = dinv * psum + 2.0 * dinv * hp_ref[...] + b_ref[...]
    if relu:
        y = jnp.maximum(y, 0.0)
    y_ref[...] = y * rm_ref[...]


def _rowwise_specs(f):
    return [
        pl.BlockSpec((2, _ROWB, f), lambda i: (0, i, 0)),
        pl.BlockSpec((_ROWB, f), lambda i: (i, 0)),
        pl.BlockSpec((_ROWB, 1), lambda i: (i, 0)),
        pl.BlockSpec((1, f), lambda i: (0, 0)),
        pl.BlockSpec((_ROWB, 1), lambda i: (i, 0)),
    ]


def _tc_finish(P, hp, dinv, b, rowmask, relu):
    n, f = hp.shape
    return pl.pallas_call(
        functools.partial(_finish_body, relu),
        grid=(n // _ROWB,),
        in_specs=_rowwise_specs(f),
        out_specs=pl.BlockSpec((_ROWB, f), lambda i: (i, 0)),
        out_shape=jax.ShapeDtypeStruct((n, f), jnp.float32),
    )(P, hp, dinv, b.reshape(1, f), rowmask)


def _logsoftmax_body(p_ref, hp_ref, dinv_ref, b_ref, cm_ref, y_ref):
    psum = p_ref[0, :, :] + p_ref[1, :, :]
    dinv = dinv_ref[...]
    y = dinv * psum + 2.0 * dinv * hp_ref[...] + b_ref[...]
    cm = cm_ref[...]
    yl = jnp.where(cm > 0, y, -1e30)
    mx = jnp.max(yl, axis=1, keepdims=True)
    ex = jnp.where(cm > 0, jnp.exp(yl - mx), 0.0)
    denom = jnp.sum(ex, axis=1, keepdims=True)
    y_ref[...] = (yl - mx) - jnp.log(denom)


def _tc_logsoftmax(P, hp, dinv, b, colmask):
    n, f = hp.shape
    specs = _rowwise_specs(f)
    specs[-1] = pl.BlockSpec((1, f), lambda i: (0, 0))
    return pl.pallas_call(
        _logsoftmax_body,
        grid=(n // _ROWB,),
        in_specs=specs,
        out_specs=pl.BlockSpec((_ROWB, f), lambda i: (i, 0)),
        out_shape=jax.ShapeDtypeStruct((n, f), jnp.float32),
    )(P, hp, dinv, b.reshape(1, f), colmask)


def _score_body(k, y_ref, p_ref, valid_ref, m_ref, s_ref):
    pv = p_ref[...]
    pn = jnp.sqrt(jnp.sum(pv * pv))
    st = jnp.tanh(jnp.dot(y_ref[...], pv,
                          preferred_element_type=jnp.float32) / pn)
    valid = valid_ref[...]
    sm = jnp.where(valid > 0, st, -3.0)
    bi = lax.bitcast_convert_type(sm, jnp.int32)
    mag = jnp.bitwise_and(bi, jnp.int32(0x7FFFFFFF))
    key = jnp.where(bi >= 0, bi, -mag)

    kf = jnp.float32(k)

    def cnt_ge(t):
        return jnp.sum(jnp.where(key >= t, 1.0, 0.0))

    def step(_, lh):
        lo, hi = lh
        mid = lo + (hi - lo + 1) // 2
        ok = cnt_ge(mid) >= kf
        return (jnp.where(ok, mid, lo), jnp.where(ok, hi, mid - 1))

    lo0 = jnp.int32(-0x40400001)
    hi0 = jnp.int32(0x3F800000)
    t, _ = lax.fori_loop(0, 32, step, (lo0, hi0))

    gt = key > t
    eq = key == t
    needed = kf - jnp.sum(jnp.where(gt, 1.0, 0.0))
    idx = lax.broadcasted_iota(jnp.int32, key.shape, 0)

    def cnt_eq_lt(T):
        return jnp.sum(jnp.where(eq & (idx < T), 1.0, 0.0))

    def step2(_, lh):
        lo, hi = lh
        mid = lo + (hi - lo + 1) // 2
        ok = cnt_eq_lt(mid) <= needed
        return (jnp.where(ok, mid, lo), jnp.where(ok, hi, mid - 1))

    T, _ = lax.fori_loop(0, 16, step2, (jnp.int32(0), jnp.int32(N_PAD)))

    sel = gt | (eq & (idx < T))
    m_ref[...] = jnp.where(sel, 1.0, 0.0)
    s_ref[...] = st


def _tc_score(Y, p, valid, k):
    n, f = Y.shape
    return pl.pallas_call(
        functools.partial(_score_body, k),
        out_shape=(jax.ShapeDtypeStruct((n, 1), jnp.float32),
                   jax.ShapeDtypeStruct((n, 1), jnp.float32)),
    )(Y, p.reshape(f, 1), valid)


def _pad_rows(a, rows):
    return jnp.pad(a, ((0, rows - a.shape[0]),) + ((0, 0),) * (a.ndim - 1))


def kernel(x, pos, edge_index, W0, b0, p1, W1, b1, p2, W2, b2,
           Wu0, bu0, Wu1, bu1):
    f32 = jnp.float32

    x0 = _pad_rows(jnp.concatenate([x, pos], axis=-1), N_PAD)
    x0 = jnp.pad(x0, ((0, 0), (0, 16 - 9)))
    W0p = jnp.pad(W0, ((0, 16 - 9), (0, 0)))
    Wu1p = jnp.pad(Wu1, ((0, 0), (0, 64 - NUM_CLASSES)))
    bu1p = jnp.pad(bu1, (0, 64 - NUM_CLASSES))

    src = jnp.full((E_PAD,), ZROW, jnp.int32).at[:E].set(edge_index[0])
    dst = jnp.full((E_PAD,), ZROW, jnp.int32).at[:E].set(edge_index[1])
    src = src.reshape(EDGE_ROWS, 128)
    dst = dst.reshape(EDGE_ROWS, 128)

    rowmask = (jnp.arange(N_PAD) < N).astype(f32).reshape(N_PAD, 1)
    z16 = jnp.zeros((N_PAD, 16), f32)
    z64 = jnp.zeros((N_PAD, 64), f32)
    z128 = jnp.zeros((N_PAD, 128), f32)
    onescale = jnp.ones((N_PAD, 1), f32)
    colmask = (jnp.arange(64) < NUM_CLASSES).astype(f32).reshape(1, 64)

    k1 = int(math.ceil(0.8 * N))
    k2 = int(math.ceil(0.8 * k1))

    agg16 = _sc_agg(16)
    agg64 = _sc_agg(64)
    agg128 = _sc_agg(128)

    def ones_table(m):
        return jnp.broadcast_to(m, (N_PAD, 16))

    deg0 = agg16(ones_table(rowmask), src, dst, z16)
    hp0, dinv0 = _tc_prep(x0, W0p, deg0, onescale, rowmask)
    P0 = agg64(hp0, src, dst, z64)
    h0 = _tc_finish(P0, hp0, dinv0, b0, rowmask, relu=True)

    m1, s1 = _tc_score(h0, p1, rowmask, k1)
    deg1 = agg16(ones_table(m1), src, dst, z16)
    hp1, dinv1 = _tc_prep(h0, W1, deg1, s1, m1)
    P1 = agg128(hp1, src, dst, z128)
    h1 = _tc_finish(P1, hp1, dinv1, b1, rowmask, relu=True)

    m2, s2 = _tc_score(h1, p2, m1, k2)
    deg2 = agg16(ones_table(m2), src, dst, z16)
    hp2, dinv2 = _tc_prep(h1, W2, deg2, s2, m2)
    P2a = agg128(hp2[:, :128], src, dst, z128)
    P2b = agg128(hp2[:, 128:], src, dst, z128)
    P2 = jnp.concatenate([P2a, P2b], axis=2)
    h2 = _tc_finish(P2, hp2, dinv2, b2, m2, relu=True)

    u1in = jnp.concatenate([h1, h2], axis=-1)
    hpu0, _ = _tc_prep(u1in, Wu0, deg1, onescale, m1)
    Pu0 = agg128(hpu0, src, dst, z128)
    u1 = _tc_finish(Pu0, hpu0, dinv1, bu0, m1, relu=True)

    oin = jnp.concatenate([h0, u1], axis=-1)
    hpu1, _ = _tc_prep(oin, Wu1p, deg0, onescale, rowmask)
    Pu1 = agg64(hpu1, src, dst, z64)
    out = _tc_logsoftmax(Pu1, hpu1, dinv0, bu1p, colmask)

    return out[:N, :NUM_CLASSES]


# speedup vs baseline: 13.8190x; 1.0064x over previous
import functools
import math

import jax
import jax.numpy as jnp
from jax import lax
from jax.experimental import pallas as pl
from jax.experimental.pallas import tpu as pltpu
from jax.experimental.pallas import tpu_sc as plsc

N = 10000
E = 160000
NUM_CLASSES = 50

N_PAD = 10240
ZROW = N
E_PAD = 163840
EDGE_ROWS = E_PAD // 128
ROWS_PER_WORKER = EDGE_ROWS // 32
STRIPE = N_PAD // 16


@functools.lru_cache(maxsize=None)
def _sc_agg(F):
    depth = 4 if F <= 64 else 2
    mesh = plsc.VectorSubcoreMesh(core_axis_name="c", subcore_axis_name="s")

    @functools.partial(
        pl.kernel,
        mesh=mesh,
        compiler_params=pltpu.CompilerParams(use_tc_tiling_on_sc=False),
        out_type=jax.ShapeDtypeStruct((2, N_PAD, F), jnp.float32),
        scratch_types=[
            pltpu.VMEM((ROWS_PER_WORKER, 128), jnp.int32),
            pltpu.VMEM((ROWS_PER_WORKER, 128), jnp.int32),
        ] + [pltpu.VMEM((128, F), jnp.float32)] * depth + [
            pltpu.VMEM_SHARED((N_PAD, F), jnp.float32),
        ] + [pltpu.SemaphoreType.DMA] * depth,
    )
    def agg(table_h, src_h, dst_h, zeros_h, out_h, srcb, dstb, *rest):
        bufs = rest[:depth]
        acc = rest[depth]
        sems = rest[depth + 1:]
        c = lax.axis_index("c")
        s = lax.axis_index("s")
        w = c * 16 + s
        pltpu.sync_copy(zeros_h.at[pl.ds(s * STRIPE, STRIPE)],
                        acc.at[pl.ds(s * STRIPE, STRIPE)])
        plsc.subcore_barrier()

        row0 = w * ROWS_PER_WORKER
        pltpu.sync_copy(src_h.at[pl.ds(row0, ROWS_PER_WORKER)], srcb)
        pltpu.sync_copy(dst_h.at[pl.ds(row0, ROWS_PER_WORKER)], dstb)

        for d in range(depth):
            pltpu.async_copy(table_h.at[srcb.at[d]], bufs[d], sems[d])

        def body(t, carry):
            j0 = depth * t
            for d in range(depth):
                j = j0 + d
                pltpu.make_async_copy(table_h.at[srcb.at[d]],
                                      bufs[d], sems[d]).wait()
                pltpu.sync_copy(bufs[d], acc.at[dstb.at[j]], add=True)

                @pl.when(j + depth < ROWS_PER_WORKER)
                def _(d=d, j=j):
                    pltpu.async_copy(table_h.at[srcb.at[j + depth]],
                                     bufs[d], sems[d])

            return carry

        lax.fori_loop(0, ROWS_PER_WORKER // depth, body, 0)
        plsc.subcore_barrier()
        pltpu.sync_copy(acc.at[pl.ds(s * STRIPE, STRIPE)],
                        out_h.at[c].at[pl.ds(s * STRIPE, STRIPE)])

    return agg


def _prep_body(x_ref, w_ref, degp_ref, scale_ref, m_ref, hp_ref, dinv_ref):
    deg = degp_ref[0, :, 0:1] + degp_ref[1, :, 0:1] + 2.0
    dinv = lax.rsqrt(deg)
    xe = x_ref[...] * scale_ref[...]
    hp = jnp.dot(xe, w_ref[...], preferred_element_type=jnp.float32) * dinv
    hp_ref[...] = hp * m_ref[...]
    dinv_ref[...] = dinv


_ROWB = 2560


def _tc_prep(X, W, degp, scale, mask):
    n, fin = X.shape
    fout = W.shape[1]
    g = n // _ROWB
    return pl.pallas_call(
        _prep_body,
        grid=(g,),
        in_specs=[
            pl.BlockSpec((_ROWB, fin), lambda i: (i, 0)),
            pl.BlockSpec((fin, fout), lambda i: (0, 0)),
            pl.BlockSpec((2, _ROWB, 16), lambda i: (0, i, 0)),
            pl.BlockSpec((_ROWB, 1), lambda i: (i, 0)),
            pl.BlockSpec((_ROWB, 1), lambda i: (i, 0)),
        ],
        out_specs=(pl.BlockSpec((_ROWB, fout), lambda i: (i, 0)),
                   pl.BlockSpec((_ROWB, 1), lambda i: (i, 0))),
        out_shape=(jax.ShapeDtypeStruct((n, fout), jnp.float32),
                   jax.ShapeDtypeStruct((n, 1), jnp.float32)),
    )(X, W, degp, scale, mask)


def _finish_body(relu, p_ref, hp_ref, dinv_ref, b_ref, rm_ref, y_ref):
    psum = p_ref[0, :, :] + p_ref[1, :, :]
    dinv = dinv_ref[...]
    y = dinv * psum + 2.0 * dinv * hp_ref[...] + b_ref[...]
    if relu:
        y = jnp.maximum(y, 0.0)
    y_ref[...] = y * rm_ref[...]


def _rowwise_specs(f):
    return [
        pl.BlockSpec((2, _ROWB, f), lambda i: (0, i, 0)),
        pl.BlockSpec((_ROWB, f), lambda i: (i, 0)),
        pl.BlockSpec((_ROWB, 1), lambda i: (i, 0)),
        pl.BlockSpec((1, f), lambda i: (0, 0)),
        pl.BlockSpec((_ROWB, 1), lambda i: (i, 0)),
    ]


def _tc_finish(P, hp, dinv, b, rowmask, relu):
    n, f = hp.shape
    return pl.pallas_call(
        functools.partial(_finish_body, relu),
        grid=(n // _ROWB,),
        in_specs=_rowwise_specs(f),
        out_specs=pl.BlockSpec((_ROWB, f), lambda i: (i, 0)),
        out_shape=jax.ShapeDtypeStruct((n, f), jnp.float32),
    )(P, hp, dinv, b.reshape(1, f), rowmask)


def _logsoftmax_body(p_ref, hp_ref, dinv_ref, b_ref, cm_ref, y_ref):
    psum = p_ref[0, :, :] + p_ref[1, :, :]
    dinv = dinv_ref[...]
    y = dinv * psum + 2.0 * dinv * hp_ref[...] + b_ref[...]
    cm = cm_ref[...]
    yl = jnp.where(cm > 0, y, -1e30)
    mx = jnp.max(yl, axis=1, keepdims=True)
    ex = jnp.where(cm > 0, jnp.exp(yl - mx), 0.0)
    denom = jnp.sum(ex, axis=1, keepdims=True)
    y_ref[...] = (yl - mx) - jnp.log(denom)


def _tc_logsoftmax(P, hp, dinv, b, colmask):
    n, f = hp.shape
    specs = _rowwise_specs(f)
    specs[-1] = pl.BlockSpec((1, f), lambda i: (0, 0))
    return pl.pallas_call(
        _logsoftmax_body,
        grid=(n // _ROWB,),
        in_specs=specs,
        out_specs=pl.BlockSpec((_ROWB, f), lambda i: (i, 0)),
        out_shape=jax.ShapeDtypeStruct((n, f), jnp.float32),
    )(P, hp, dinv, b.reshape(1, f), colmask)


def _score_body(k, y_ref, p_ref, valid_ref, m_ref, s_ref):
    pv = p_ref[...]
    pn = jnp.sqrt(jnp.sum(pv * pv))
    st = jnp.tanh(jnp.dot(y_ref[...], pv,
                          preferred_element_type=jnp.float32) / pn)
    valid = valid_ref[...]
    sm = jnp.where(valid > 0, st, -3.0)
    bi = lax.bitcast_convert_type(sm, jnp.int32)
    mag = jnp.bitwise_and(bi, jnp.int32(0x7FFFFFFF))
    key = jnp.where(bi >= 0, bi, -mag)

    kf = jnp.float32(k)

    def cnt_ge(t):
        return jnp.sum(jnp.where(key >= t, 1.0, 0.0))

    def step(_, lh):
        lo, hi = lh
        mid = lo + (hi - lo + 1) // 2
        ok = cnt_ge(mid) >= kf
        return (jnp.where(ok, mid, lo), jnp.where(ok, hi, mid - 1))

    lo0 = jnp.int32(-0x40400001)
    hi0 = jnp.int32(0x3F800000)
    t, _ = lax.fori_loop(0, 32, step, (lo0, hi0))

    gt = key > t
    eq = key == t
    needed = kf - jnp.sum(jnp.where(gt, 1.0, 0.0))
    idx = lax.broadcasted_iota(jnp.int32, key.shape, 0)

    def cnt_eq_lt(T):
        return jnp.sum(jnp.where(eq & (idx < T), 1.0, 0.0))

    def step2(_, lh):
        lo, hi = lh
        mid = lo + (hi - lo + 1) // 2
        ok = cnt_eq_lt(mid) <= needed
        return (jnp.where(ok, mid, lo), jnp.where(ok, hi, mid - 1))

    T, _ = lax.fori_loop(0, 16, step2, (jnp.int32(0), jnp.int32(N_PAD)))

    sel = gt | (eq & (idx < T))
    m_ref[...] = jnp.where(sel, 1.0, 0.0)
    s_ref[...] = st


def _tc_score(Y, p, valid, k):
    n, f = Y.shape
    return pl.pallas_call(
        functools.partial(_score_body, k),
        out_shape=(jax.ShapeDtypeStruct((n, 1), jnp.float32),
                   jax.ShapeDtypeStruct((n, 1), jnp.float32)),
    )(Y, p.reshape(f, 1), valid)


def _pad_rows(a, rows):
    return jnp.pad(a, ((0, rows - a.shape[0]),) + ((0, 0),) * (a.ndim - 1))


def kernel(x, pos, edge_index, W0, b0, p1, W1, b1, p2, W2, b2,
           Wu0, bu0, Wu1, bu1):
    f32 = jnp.float32

    x0 = _pad_rows(jnp.concatenate([x, pos], axis=-1), N_PAD)
    x0 = jnp.pad(x0, ((0, 0), (0, 16 - 9)))
    W0p = jnp.pad(W0, ((0, 16 - 9), (0, 0)))
    Wu1p = jnp.pad(Wu1, ((0, 0), (0, 64 - NUM_CLASSES)))
    bu1p = jnp.pad(bu1, (0, 64 - NUM_CLASSES))

    src = jnp.full((E_PAD,), ZROW, jnp.int32).at[:E].set(edge_index[0])
    dst = jnp.full((E_PAD,), ZROW, jnp.int32).at[:E].set(edge_index[1])
    src = src.reshape(EDGE_ROWS, 128)
    dst = dst.reshape(EDGE_ROWS, 128)

    rowmask = (jnp.arange(N_PAD) < N).astype(f32).reshape(N_PAD, 1)
    z16 = jnp.zeros((N_PAD, 16), f32)
    z64 = jnp.zeros((N_PAD, 64), f32)
    z128 = jnp.zeros((N_PAD, 128), f32)
    onescale = jnp.ones((N_PAD, 1), f32)
    colmask = (jnp.arange(64) < NUM_CLASSES).astype(f32).reshape(1, 64)

    k1 = int(math.ceil(0.8 * N))
    k2 = int(math.ceil(0.8 * k1))

    agg16 = _sc_agg(16)
    agg64 = _sc_agg(64)
    agg128 = _sc_agg(128)

    def ones_table(m):
        return jnp.broadcast_to(m, (N_PAD, 16))

    deg0 = agg16(ones_table(rowmask), src, dst, z16)
    hp0, dinv0 = _tc_prep(x0, W0p, deg0, onescale, rowmask)
    P0 = agg64(hp0, src, dst, z64)
    h0 = _tc_finish(P0, hp0, dinv0, b0, rowmask, relu=True)

    m1, s1 = _tc_score(h0, p1, rowmask, k1)
    deg1 = agg16(ones_table(m1), src, dst, z16)
    hp1, dinv1 = _tc_prep(h0, W1, deg1, s1, m1)
    P1 = agg128(hp1, src, dst, z128)
    h1 = _tc_finish(P1, hp1, dinv1, b1, rowmask, relu=True)

    m2, s2 = _tc_score(h1, p2, m1, k2)
    deg2 = agg16(ones_table(m2), src, dst, z16)
    hp2, dinv2 = _tc_prep(h1, W2, deg2, s2, m2)
    P2a = agg128(hp2[:, :128], src, dst, z128)
    P2b = agg128(hp2[:, 128:], src, dst, z128)
    P2 = jnp.concatenate([P2a, P2b], axis=2)
    h2 = _tc_finish(P2, hp2, dinv2, b2, m2, relu=True)

    u1in = jnp.concatenate([h1, h2], axis=-1)
    hpu0, _ = _tc_prep(u1in, Wu0, deg1, onescale, m1)
    Pu0 = agg128(hpu0, src, dst, z128)
    u1 = _tc_finish(Pu0, hpu0, dinv1, bu0, m1, relu=True)

    oin = jnp.concatenate([h0, u1], axis=-1)
    hpu1, _ = _tc_prep(oin, Wu1p, deg0, onescale, rowmask)
    Pu1 = agg64(hpu1, src, dst, z64)
    out = _tc_logsoftmax(Pu1, hpu1, dinv0, bu1p, colmask)

    return out[:N, :NUM_CLASSES]
